# baseline (speedup 1.0000x reference)
import jax
import jax.numpy as jnp
import numpy as np
from jax import lax
from jax.experimental import pallas as pl
from jax.experimental.pallas import tpu as pltpu
from jax.experimental.pallas import tpu_sc as plsc

D_MODEL = 256
N_HEADS = 8
N_LEVELS = 4
N_POINTS = 4
HEAD_DIM = 32
LEVEL_WH = (48, 24, 12, 6)
LEVEL_OFF = (0, 2304, 2880, 3024)
LEN_V = 3060
NQ_PAD = 3072
BQ = 512
NQB = NQ_PAD // BQ
CH = 64
N_CMP = 4
LP = N_LEVELS * N_POINTS
N_CHUNK = (NQ_PAD // 2) // CH
N_OUTER = N_CHUNK // 2

_INTERPRET = False


def _lane_consts():
    k = np.arange(N_HEADS * LP)
    lvl = (k // N_POINTS) % N_LEVELS
    w = np.asarray(LEVEL_WH, np.float32)[lvl]
    base = np.asarray(LEVEL_OFF, np.float32)[lvl]
    bx = (np.arange(8)[:, None] == (lvl * 2)[None, :]).astype(np.float32)
    by = (np.arange(8)[:, None] == (lvl * 2 + 1)[None, :]).astype(np.float32)
    g = (k[:, None] // LP == k[None, :] // LP).astype(np.float32)
    return w, base, bx, by, g


def _prep_body(q_ref, rb_ref, val_ref, wcat_ref, waw_ref, wv_ref,
               bcat_ref, baw_ref, bv_ref, bxy_ref, g_ref, wvec_ref, base_ref,
               lo_ref, hi_ref, v_ref, pk_ref):
    f32 = jnp.float32
    i32 = jnp.int32
    hp = lax.Precision.HIGHEST
    q = q_ref[0]
    big = jnp.dot(q, wcat_ref[...], preferred_element_type=f32, precision=hp) + bcat_ref[0]
    logits = jnp.dot(q, waw_ref[...], preferred_element_type=f32) + baw_ref[0]
    rb = rb_ref[0]
    rbxy = jnp.dot(rb, bxy_ref[...], preferred_element_type=f32, precision=hp)
    e = jnp.exp(logits)
    s = jnp.dot(e, g_ref[...], preferred_element_type=f32, precision=hp)
    aw = e / s
    wv = wvec_ref[0]
    basev = base_ref[0]
    gx = rbxy[:, 0:128] + big[:, 0:128]
    gy = rbxy[:, 128:256] + big[:, 128:256]
    x0 = jnp.floor(gx)
    y0 = jnp.floor(gy)
    fx = gx - x0
    fy = gy - y0
    wm1 = wv - 1.0
    vx0 = ((x0 >= 0.0) & (x0 <= wm1)).astype(i32)
    vx1 = ((x0 >= -1.0) & (x0 <= wm1 - 1.0)).astype(i32)
    vy0 = ((y0 >= 0.0) & (y0 <= wm1)).astype(i32)
    vy1 = ((y0 >= -1.0) & (y0 <= wm1 - 1.0)).astype(i32)
    xc = jnp.clip(x0, 0.0, wm1)
    yc = jnp.clip(y0, 0.0, wm1)
    dx = jnp.clip((jnp.clip(x0 + 1.0, 0.0, wm1) - xc).astype(i32), 0, 1)
    dy = jnp.clip((jnp.clip(y0 + 1.0, 0.0, wm1) - yc).astype(i32), 0, 1)
    idx_i = jnp.clip((basev + yc * wv + xc).astype(i32), lo_ref[0], hi_ref[0])
    packed = (idx_i | (vx0 << 12) | (vx1 << 13) | (vy0 << 14) | (vy1 << 15)
              | (dx << 16) | (dy << 17))
    comps = (packed, lax.bitcast_convert_type(fx, i32),
             lax.bitcast_convert_type(fy, i32),
             lax.bitcast_convert_type(aw, i32))
    for cc, arr in enumerate(comps):
        for h in range(N_HEADS):
            pk_ref[0, h, :, pl.ds(cc * LP, LP)] = arr[:, h * LP:(h + 1) * LP]
    v = jnp.dot(val_ref[0], wv_ref[...], preferred_element_type=f32) + bv_ref[0]
    for h in range(N_HEADS):
        v_ref[0, h] = v[:, h * HEAD_DIM:(h + 1) * HEAD_DIM]


def _prep_call(bs, q_p, rb8, val_p, wcat, waw, wv, bcat, baw, bv, bxy, g,
               wvec, basev, lo_i, hi_i):
    grid = (bs, NQB)
    qmap = lambda b, i: (b, i, 0)
    full = lambda b, i: (0, 0)
    return pl.pallas_call(
        _prep_body,
        grid=grid,
        in_specs=[
            pl.BlockSpec((1, BQ, D_MODEL), qmap),
            pl.BlockSpec((1, BQ, 8), qmap),
            pl.BlockSpec((1, BQ, D_MODEL), qmap),
            pl.BlockSpec((D_MODEL, 256), full),
            pl.BlockSpec((D_MODEL, 128), full),
            pl.BlockSpec((D_MODEL, D_MODEL), full),
            pl.BlockSpec((1, 256), full),
            pl.BlockSpec((1, 128), full),
            pl.BlockSpec((1, D_MODEL), full),
            pl.BlockSpec((8, 256), full),
            pl.BlockSpec((128, 128), full),
            pl.BlockSpec((1, 128), full),
            pl.BlockSpec((1, 128), full),
            pl.BlockSpec((1, 128), full),
            pl.BlockSpec((1, 128), full),
        ],
        out_specs=[
            pl.BlockSpec((1, N_HEADS, BQ, HEAD_DIM), lambda b, i: (b, 0, i, 0)),
            pl.BlockSpec((1, N_HEADS, BQ, N_CMP * LP), lambda b, i: (b, 0, i, 0)),
        ],
        out_shape=[
            jax.ShapeDtypeStruct((bs, N_HEADS, NQ_PAD, HEAD_DIM), jnp.float32),
            jax.ShapeDtypeStruct((bs, N_HEADS, NQ_PAD, N_CMP * LP), jnp.int32),
        ],
        interpret=_INTERPRET,
    )(q_p, rb8, val_p, wcat, waw, wv, bcat, baw, bv, bxy, g, wvec, basev,
      lo_i, hi_i)


def _lane_bcast(vec, t):
    tv = jnp.full((16,), t, jnp.int32)
    dn = lax.GatherDimensionNumbers(offset_dims=(), collapsed_slice_dims=(0,),
                                    start_index_map=(0,))
    return lax.gather(vec, tv[:, None], dimension_numbers=dn, slice_sizes=(1,),
                      mode=lax.GatherScatterMode.PROMISE_IN_BOUNDS)


_W32 = tuple(int(w) * HEAD_DIM for w in LEVEL_WH)


def _sc_body(table_hbm, pk_hbm, out_hbm, table_v, in_v, out_v,
             sem_t, sem_i0, sem_i1, sem_o0, sem_o1):
    c = lax.axis_index("c")
    s = lax.axis_index("s")
    wid = s * 2 + c
    bh = wid % 16
    half = wid // 16
    iota_lo = lax.iota(jnp.int32, 16)
    qbase = half * (NQ_PAD // 2)
    sem_i = (sem_i0, sem_i1)
    sem_o = (sem_o0, sem_o1)
    lvl = lax.iota(jnp.int32, 16) >> 2
    w32 = jnp.where(lvl == 0, _W32[0],
                    jnp.where(lvl == 1, _W32[1],
                              jnp.where(lvl == 2, _W32[2], _W32[3])))

    def in_copy(k, b):
        q0 = qbase + k * CH
        return pltpu.make_async_copy(
            pk_hbm.at[bh, pl.ds(q0, CH), :], in_v.at[b], sem_i[b])

    def out_copy(k, b):
        q0 = qbase + k * CH
        return pltpu.make_async_copy(
            out_v.at[b], out_hbm.at[bh, pl.ds(q0, CH), :], sem_o[b])

    tdesc = pltpu.make_async_copy(table_hbm.at[bh], table_v, sem_t)
    tdesc.start()
    for b in (0, 1):
        in_copy(b, b).start()
    tdesc.wait()

    def outer(ci2, carry):
        for b in (0, 1):
            k = ci2 * 2 + b
            in_copy(k, b).wait()

            @pl.when(ci2 > 0)
            def _():
                out_copy(k - 2, b).wait()

            def qloop(qi, carry2):
                idxf = in_v[b, qi, pl.ds(0, LP)]
                fx = plsc.bitcast(in_v[b, qi, pl.ds(LP, LP)], jnp.float32)
                fy = plsc.bitcast(in_v[b, qi, pl.ds(2 * LP, LP)], jnp.float32)
                aw = plsc.bitcast(in_v[b, qi, pl.ds(3 * LP, LP)], jnp.float32)
                rows_a = (idxf & 0xFFF) * HEAD_DIM
                mx0 = ((idxf >> 12) & 1).astype(jnp.float32)
                mx1 = ((idxf >> 13) & 1).astype(jnp.float32)
                my0 = ((idxf >> 14) & 1).astype(jnp.float32)
                my1 = ((idxf >> 15) & 1).astype(jnp.float32)
                ax = (1.0 - fx) * mx0
                bxw = fx * mx1
                ay = (1.0 - fy) * my0 * aw
                byw = fy * my1 * aw
                dxw = ((idxf >> 16) & 1) << 5
                rows_c = rows_a + ((idxf >> 17) & 1) * w32
                groups = ((rows_a, ax * ay),
                          (rows_a + dxw, bxw * ay),
                          (rows_c, ax * byw),
                          (rows_c + dxw, bxw * byw))
                accs = []
                for gpair in (groups[:2], groups[2:]):
                    acc_lo = jnp.zeros((16,), jnp.float32)
                    acc_hi = jnp.zeros((16,), jnp.float32)
                    for rows, ws in gpair:
                        for t in range(16):
                            addr_lo = _lane_bcast(rows, t) + iota_lo
                            wb = _lane_bcast(ws, t)
                            lo = plsc.load_gather(table_v, [addr_lo])
                            hi = plsc.load_gather(table_v, [addr_lo + 16])
                            acc_lo = acc_lo + lo * wb
                            acc_hi = acc_hi + hi * wb
                    accs.append((acc_lo, acc_hi))
                (a0, h0), (a1, h1) = accs
                out_v[b, qi, pl.ds(0, 16)] = a0 + a1
                out_v[b, qi, pl.ds(16, 16)] = h0 + h1
                return carry2

            lax.fori_loop(0, CH, qloop, 0)
            out_copy(k, b).start()

            @pl.when(ci2 < N_OUTER - 1)
            def _():
                in_copy(k + 2, b).start()
        return carry

    lax.fori_loop(0, N_OUTER, outer, 0)
    for b in (0, 1):
        out_copy(N_CHUNK - 2 + b, b).wait()


def _sc_call(tables, packed):
    bs_h = tables.shape[0]
    mesh = plsc.VectorSubcoreMesh(core_axis_name="c", subcore_axis_name="s")
    f = pl.kernel(
        _sc_body,
        out_type=jax.ShapeDtypeStruct((bs_h, NQ_PAD, HEAD_DIM), jnp.float32),
        mesh=mesh,
        scratch_types=[
            pltpu.VMEM((NQ_PAD * HEAD_DIM,), jnp.float32),
            pltpu.VMEM((2, CH, N_CMP * LP), jnp.int32),
            pltpu.VMEM((2, CH, HEAD_DIM), jnp.float32),
            pltpu.SemaphoreType.DMA,
            pltpu.SemaphoreType.DMA,
            pltpu.SemaphoreType.DMA,
            pltpu.SemaphoreType.DMA,
            pltpu.SemaphoreType.DMA,
        ],
        compiler_params=pltpu.CompilerParams(needs_layout_passes=False),
    )
    return f(tables, packed)


def _proj_body(x_ref, w_ref, b_ref, o_ref):
    acc = jnp.broadcast_to(b_ref[0], (BQ, D_MODEL))
    for h in range(N_HEADS):
        acc = acc + jnp.dot(x_ref[0, h], w_ref[h],
                            preferred_element_type=jnp.float32)
    o_ref[0] = acc


def _proj_call(x, w, b):
    bs = x.shape[0]
    return pl.pallas_call(
        _proj_body,
        grid=(bs, NQB),
        in_specs=[
            pl.BlockSpec((1, N_HEADS, BQ, HEAD_DIM), lambda bb, i: (bb, 0, i, 0)),
            pl.BlockSpec((N_HEADS, HEAD_DIM, D_MODEL), lambda bb, i: (0, 0, 0)),
            pl.BlockSpec((1, D_MODEL), lambda bb, i: (0, 0)),
        ],
        out_specs=pl.BlockSpec((1, BQ, D_MODEL), lambda bb, i: (bb, i, 0)),
        out_shape=jax.ShapeDtypeStruct((bs, LEN_V, D_MODEL), jnp.float32),
        interpret=_INTERPRET,
    )(x, w, b)


def kernel(query, refer_bbox, value, value_shapes, W_so, b_so, W_aw, b_aw,
           W_v, b_v, W_o, b_o):
    bs, len_q, _ = query.shape
    rb8 = refer_bbox.reshape(bs, len_q, 8)

    wnp, basenp, bxnp, bynp, gnp = _lane_consts()
    wcat = jnp.concatenate([W_so[:, 0::2], W_so[:, 1::2]], axis=1)
    bcat = jnp.concatenate([b_so[0::2] - 0.5, b_so[1::2] - 0.5])[None]
    bxy = jnp.asarray(np.concatenate([bxnp * wnp, bynp * wnp], axis=1))
    wvec = jnp.asarray(wnp)[None]
    basev = jnp.asarray(basenp)[None]
    g = jnp.asarray(gnp)
    lo_i = jnp.asarray(basenp.astype(np.int32))[None]
    hi_i = jnp.asarray((basenp + wnp * wnp - 1).astype(np.int32))[None]

    v, pk = _prep_call(bs, query, rb8, value, wcat, W_aw, W_v,
                       bcat, b_aw[None], b_v[None], bxy, g, wvec, basev,
                       lo_i, hi_i)

    tables = v.reshape(bs * N_HEADS, NQ_PAD * HEAD_DIM)
    out_sc = _sc_call(tables, pk.reshape(bs * N_HEADS, NQ_PAD, N_CMP * LP))

    attn = out_sc.reshape(bs, N_HEADS, NQ_PAD, HEAD_DIM)
    return _proj_call(attn, W_o.reshape(N_HEADS, HEAD_DIM, D_MODEL), b_o[None])


# speedup vs baseline: 1.0139x; 1.0139x over previous
import jax
import jax.numpy as jnp
import numpy as np
from jax import lax
from jax.experimental import pallas as pl
from jax.experimental.pallas import tpu as pltpu
from jax.experimental.pallas import tpu_sc as plsc

D_MODEL = 256
N_HEADS = 8
N_LEVELS = 4
N_POINTS = 4
HEAD_DIM = 32
LEVEL_WH = (48, 24, 12, 6)
LEVEL_OFF = (0, 2304, 2880, 3024)
LEN_V = 3060
NQ_PAD = 3072
BQ = 1024
NQB = NQ_PAD // BQ
CH = 64
N_CMP = 4
LP = N_LEVELS * N_POINTS
N_CHUNK = (NQ_PAD // 2) // CH
N_OUTER = N_CHUNK // 2

_INTERPRET = False


def _lane_consts():
    k = np.arange(N_HEADS * LP)
    lvl = (k // N_POINTS) % N_LEVELS
    w = np.asarray(LEVEL_WH, np.float32)[lvl]
    base = np.asarray(LEVEL_OFF, np.float32)[lvl]
    bx = (np.arange(8)[:, None] == (lvl * 2)[None, :]).astype(np.float32)
    by = (np.arange(8)[:, None] == (lvl * 2 + 1)[None, :]).astype(np.float32)
    g = (k[:, None] // LP == k[None, :] // LP).astype(np.float32)
    return w, base, bx, by, g


def _prep_body(q_ref, rb_ref, val_ref, wcat_ref, waw_ref, wv_ref,
               bcat_ref, baw_ref, bv_ref, bxy_ref, g_ref, wvec_ref, base_ref,
               lo_ref, hi_ref, v_ref, pk_ref):
    f32 = jnp.float32
    i32 = jnp.int32
    hp = lax.Precision.HIGHEST
    q = q_ref[0]
    big = jnp.dot(q, wcat_ref[...], preferred_element_type=f32, precision=hp) + bcat_ref[0]
    logits = jnp.dot(q, waw_ref[...], preferred_element_type=f32) + baw_ref[0]
    rb = rb_ref[0]
    rbxy = jnp.dot(rb, bxy_ref[...], preferred_element_type=f32, precision=hp)
    e = jnp.exp(logits)
    s = jnp.dot(e, g_ref[...], preferred_element_type=f32, precision=hp)
    aw = e / s
    wv = wvec_ref[0]
    basev = base_ref[0]
    gx = rbxy[:, 0:128] + big[:, 0:128]
    gy = rbxy[:, 128:256] + big[:, 128:256]
    x0 = jnp.floor(gx)
    y0 = jnp.floor(gy)
    fx = gx - x0
    fy = gy - y0
    wm1 = wv - 1.0
    vx0 = ((x0 >= 0.0) & (x0 <= wm1)).astype(i32)
    vx1 = ((x0 >= -1.0) & (x0 <= wm1 - 1.0)).astype(i32)
    vy0 = ((y0 >= 0.0) & (y0 <= wm1)).astype(i32)
    vy1 = ((y0 >= -1.0) & (y0 <= wm1 - 1.0)).astype(i32)
    xc = jnp.clip(x0, 0.0, wm1)
    yc = jnp.clip(y0, 0.0, wm1)
    dx = jnp.clip((jnp.clip(x0 + 1.0, 0.0, wm1) - xc).astype(i32), 0, 1)
    dy = jnp.clip((jnp.clip(y0 + 1.0, 0.0, wm1) - yc).astype(i32), 0, 1)
    idx_i = jnp.clip((basev + yc * wv + xc).astype(i32), lo_ref[0], hi_ref[0])
    packed = (idx_i | (vx0 << 12) | (vx1 << 13) | (vy0 << 14) | (vy1 << 15)
              | (dx << 16) | (dy << 17))
    comps = (packed, lax.bitcast_convert_type(fx, i32),
             lax.bitcast_convert_type(fy, i32),
             lax.bitcast_convert_type(aw, i32))
    for cc, arr in enumerate(comps):
        for h in range(N_HEADS):
            pk_ref[0, h, :, pl.ds(cc * LP, LP)] = arr[:, h * LP:(h + 1) * LP]
    v = jnp.dot(val_ref[0], wv_ref[...], preferred_element_type=f32) + bv_ref[0]
    for h in range(N_HEADS):
        v_ref[0, h] = v[:, h * HEAD_DIM:(h + 1) * HEAD_DIM]


def _prep_call(bs, q_p, rb8, val_p, wcat, waw, wv, bcat, baw, bv, bxy, g,
               wvec, basev, lo_i, hi_i):
    grid = (bs, NQB)
    qmap = lambda b, i: (b, i, 0)
    full = lambda b, i: (0, 0)
    return pl.pallas_call(
        _prep_body,
        grid=grid,
        in_specs=[
            pl.BlockSpec((1, BQ, D_MODEL), qmap),
            pl.BlockSpec((1, BQ, 8), qmap),
            pl.BlockSpec((1, BQ, D_MODEL), qmap),
            pl.BlockSpec((D_MODEL, 256), full),
            pl.BlockSpec((D_MODEL, 128), full),
            pl.BlockSpec((D_MODEL, D_MODEL), full),
            pl.BlockSpec((1, 256), full),
            pl.BlockSpec((1, 128), full),
            pl.BlockSpec((1, D_MODEL), full),
            pl.BlockSpec((8, 256), full),
            pl.BlockSpec((128, 128), full),
            pl.BlockSpec((1, 128), full),
            pl.BlockSpec((1, 128), full),
            pl.BlockSpec((1, 128), full),
            pl.BlockSpec((1, 128), full),
        ],
        out_specs=[
            pl.BlockSpec((1, N_HEADS, BQ, HEAD_DIM), lambda b, i: (b, 0, i, 0)),
            pl.BlockSpec((1, N_HEADS, BQ, N_CMP * LP), lambda b, i: (b, 0, i, 0)),
        ],
        out_shape=[
            jax.ShapeDtypeStruct((bs, N_HEADS, NQ_PAD, HEAD_DIM), jnp.float32),
            jax.ShapeDtypeStruct((bs, N_HEADS, NQ_PAD, N_CMP * LP), jnp.int32),
        ],
        interpret=_INTERPRET,
    )(q_p, rb8, val_p, wcat, waw, wv, bcat, baw, bv, bxy, g, wvec, basev,
      lo_i, hi_i)


def _lane_bcast(vec, t):
    tv = jnp.full((16,), t, jnp.int32)
    dn = lax.GatherDimensionNumbers(offset_dims=(), collapsed_slice_dims=(0,),
                                    start_index_map=(0,))
    return lax.gather(vec, tv[:, None], dimension_numbers=dn, slice_sizes=(1,),
                      mode=lax.GatherScatterMode.PROMISE_IN_BOUNDS)


_W32 = tuple(int(w) * HEAD_DIM for w in LEVEL_WH)


def _sc_body(table_hbm, pk_hbm, out_hbm, table_v, in_v, out_v,
             sem_t, sem_i0, sem_i1, sem_o0, sem_o1):
    c = lax.axis_index("c")
    s = lax.axis_index("s")
    wid = s * 2 + c
    bh = wid % 16
    half = wid // 16
    iota_lo = lax.iota(jnp.int32, 16)
    qbase = half * (NQ_PAD // 2)
    sem_i = (sem_i0, sem_i1)
    sem_o = (sem_o0, sem_o1)
    lvl = lax.iota(jnp.int32, 16) >> 2
    w32 = jnp.where(lvl == 0, _W32[0],
                    jnp.where(lvl == 1, _W32[1],
                              jnp.where(lvl == 2, _W32[2], _W32[3])))

    def in_copy(k, b):
        q0 = qbase + k * CH
        return pltpu.make_async_copy(
            pk_hbm.at[bh, pl.ds(q0, CH), :], in_v.at[b], sem_i[b])

    def out_copy(k, b):
        q0 = qbase + k * CH
        return pltpu.make_async_copy(
            out_v.at[b], out_hbm.at[bh, pl.ds(q0, CH), :], sem_o[b])

    tdesc = pltpu.make_async_copy(table_hbm.at[bh], table_v, sem_t)
    tdesc.start()
    for b in (0, 1):
        in_copy(b, b).start()
    tdesc.wait()

    def outer(ci2, carry):
        for b in (0, 1):
            k = ci2 * 2 + b
            in_copy(k, b).wait()

            @pl.when(ci2 > 0)
            def _():
                out_copy(k - 2, b).wait()

            def qloop(qi, carry2):
                idxf = in_v[b, qi, pl.ds(0, LP)]
                fx = plsc.bitcast(in_v[b, qi, pl.ds(LP, LP)], jnp.float32)
                fy = plsc.bitcast(in_v[b, qi, pl.ds(2 * LP, LP)], jnp.float32)
                aw = plsc.bitcast(in_v[b, qi, pl.ds(3 * LP, LP)], jnp.float32)
                rows_a = (idxf & 0xFFF) * HEAD_DIM
                mx0 = ((idxf >> 12) & 1).astype(jnp.float32)
                mx1 = ((idxf >> 13) & 1).astype(jnp.float32)
                my0 = ((idxf >> 14) & 1).astype(jnp.float32)
                my1 = ((idxf >> 15) & 1).astype(jnp.float32)
                ax = (1.0 - fx) * mx0
                bxw = fx * mx1
                ay = (1.0 - fy) * my0 * aw
                byw = fy * my1 * aw
                dxw = ((idxf >> 16) & 1) << 5
                rows_c = rows_a + ((idxf >> 17) & 1) * w32
                groups = ((rows_a, ax * ay),
                          (rows_a + dxw, bxw * ay),
                          (rows_c, ax * byw),
                          (rows_c + dxw, bxw * byw))
                accs = []
                for gpair in (groups[:2], groups[2:]):
                    acc_lo = jnp.zeros((16,), jnp.float32)
                    acc_hi = jnp.zeros((16,), jnp.float32)
                    for rows, ws in gpair:
                        for t in range(16):
                            addr_lo = _lane_bcast(rows, t) + iota_lo
                            wb = _lane_bcast(ws, t)
                            lo = plsc.load_gather(table_v, [addr_lo])
                            hi = plsc.load_gather(table_v, [addr_lo + 16])
                            acc_lo = acc_lo + lo * wb
                            acc_hi = acc_hi + hi * wb
                    accs.append((acc_lo, acc_hi))
                (a0, h0), (a1, h1) = accs
                out_v[b, qi, pl.ds(0, 16)] = a0 + a1
                out_v[b, qi, pl.ds(16, 16)] = h0 + h1
                return carry2

            lax.fori_loop(0, CH, qloop, 0)
            out_copy(k, b).start()

            @pl.when(ci2 < N_OUTER - 1)
            def _():
                in_copy(k + 2, b).start()
        return carry

    lax.fori_loop(0, N_OUTER, outer, 0)
    for b in (0, 1):
        out_copy(N_CHUNK - 2 + b, b).wait()


def _sc_call(tables, packed):
    bs_h = tables.shape[0]
    mesh = plsc.VectorSubcoreMesh(core_axis_name="c", subcore_axis_name="s")
    f = pl.kernel(
        _sc_body,
        out_type=jax.ShapeDtypeStruct((bs_h, NQ_PAD, HEAD_DIM), jnp.float32),
        mesh=mesh,
        scratch_types=[
            pltpu.VMEM((NQ_PAD * HEAD_DIM,), jnp.float32),
            pltpu.VMEM((2, CH, N_CMP * LP), jnp.int32),
            pltpu.VMEM((2, CH, HEAD_DIM), jnp.float32),
            pltpu.SemaphoreType.DMA,
            pltpu.SemaphoreType.DMA,
            pltpu.SemaphoreType.DMA,
            pltpu.SemaphoreType.DMA,
            pltpu.SemaphoreType.DMA,
        ],
        compiler_params=pltpu.CompilerParams(needs_layout_passes=False),
    )
    return f(tables, packed)


def _proj_body(x_ref, w_ref, b_ref, o_ref):
    acc = jnp.broadcast_to(b_ref[0], (BQ, D_MODEL))
    for h in range(N_HEADS):
        acc = acc + jnp.dot(x_ref[0, h], w_ref[h],
                            preferred_element_type=jnp.float32)
    o_ref[0] = acc


def _proj_call(x, w, b):
    bs = x.shape[0]
    return pl.pallas_call(
        _proj_body,
        grid=(bs, NQB),
        in_specs=[
            pl.BlockSpec((1, N_HEADS, BQ, HEAD_DIM), lambda bb, i: (bb, 0, i, 0)),
            pl.BlockSpec((N_HEADS, HEAD_DIM, D_MODEL), lambda bb, i: (0, 0, 0)),
            pl.BlockSpec((1, D_MODEL), lambda bb, i: (0, 0)),
        ],
        out_specs=pl.BlockSpec((1, BQ, D_MODEL), lambda bb, i: (bb, i, 0)),
        out_shape=jax.ShapeDtypeStruct((bs, LEN_V, D_MODEL), jnp.float32),
        interpret=_INTERPRET,
    )(x, w, b)


def kernel(query, refer_bbox, value, value_shapes, W_so, b_so, W_aw, b_aw,
           W_v, b_v, W_o, b_o):
    bs, len_q, _ = query.shape
    rb8 = refer_bbox.reshape(bs, len_q, 8)

    wnp, basenp, bxnp, bynp, gnp = _lane_consts()
    wcat = jnp.concatenate([W_so[:, 0::2], W_so[:, 1::2]], axis=1)
    bcat = jnp.concatenate([b_so[0::2] - 0.5, b_so[1::2] - 0.5])[None]
    bxy = jnp.asarray(np.concatenate([bxnp * wnp, bynp * wnp], axis=1))
    wvec = jnp.asarray(wnp)[None]
    basev = jnp.asarray(basenp)[None]
    g = jnp.asarray(gnp)
    lo_i = jnp.asarray(basenp.astype(np.int32))[None]
    hi_i = jnp.asarray((basenp + wnp * wnp - 1).astype(np.int32))[None]

    v, pk = _prep_call(bs, query, rb8, value, wcat, W_aw, W_v,
                       bcat, b_aw[None], b_v[None], bxy, g, wvec, basev,
                       lo_i, hi_i)

    tables = v.reshape(bs * N_HEADS, NQ_PAD * HEAD_DIM)
    out_sc = _sc_call(tables, pk.reshape(bs * N_HEADS, NQ_PAD, N_CMP * LP))

    attn = out_sc.reshape(bs, N_HEADS, NQ_PAD, HEAD_DIM)
    return _proj_call(attn, W_o.reshape(N_HEADS, HEAD_DIM, D_MODEL), b_o[None])


# speedup vs baseline: 1.0146x; 1.0007x over previous
import jax
import jax.numpy as jnp
import numpy as np
from jax import lax
from jax.experimental import pallas as pl
from jax.experimental.pallas import tpu as pltpu
from jax.experimental.pallas import tpu_sc as plsc

D_MODEL = 256
N_HEADS = 8
N_LEVELS = 4
N_POINTS = 4
HEAD_DIM = 32
LEVEL_WH = (48, 24, 12, 6)
LEVEL_OFF = (0, 2304, 2880, 3024)
LEN_V = 3060
NQ_PAD = 3072
BQ = 1024
NQB = NQ_PAD // BQ
CH = 64
N_CMP = 4
LP = N_LEVELS * N_POINTS
N_CHUNK = (NQ_PAD // 2) // CH
N_OUTER = N_CHUNK // 2

def _lane_consts():
    k = np.arange(N_HEADS * LP)
    lvl = (k // N_POINTS) % N_LEVELS
    w = np.asarray(LEVEL_WH, np.float32)[lvl]
    base = np.asarray(LEVEL_OFF, np.float32)[lvl]
    bx = (np.arange(8)[:, None] == (lvl * 2)[None, :]).astype(np.float32)
    by = (np.arange(8)[:, None] == (lvl * 2 + 1)[None, :]).astype(np.float32)
    g = (k[:, None] // LP == k[None, :] // LP).astype(np.float32)
    return w, base, bx, by, g


def _prep_body(q_ref, rb_ref, val_ref, wcat_ref, waw_ref, wv_ref,
               bcat_ref, baw_ref, bv_ref, bxy_ref, g_ref, wvec_ref, base_ref,
               lo_ref, hi_ref, v_ref, pk_ref):
    f32 = jnp.float32
    i32 = jnp.int32
    hp = lax.Precision.HIGHEST
    q = q_ref[0]
    big = jnp.dot(q, wcat_ref[...], preferred_element_type=f32, precision=hp) + bcat_ref[0]
    logits = jnp.dot(q, waw_ref[...], preferred_element_type=f32) + baw_ref[0]
    rb = rb_ref[0]
    rbxy = jnp.dot(rb, bxy_ref[...], preferred_element_type=f32, precision=hp)
    e = jnp.exp(logits)
    s = jnp.dot(e, g_ref[...], preferred_element_type=f32, precision=hp)
    aw = e / s
    wv = wvec_ref[0]
    basev = base_ref[0]
    gx = rbxy[:, 0:128] + big[:, 0:128]
    gy = rbxy[:, 128:256] + big[:, 128:256]
    x0 = jnp.floor(gx)
    y0 = jnp.floor(gy)
    fx = gx - x0
    fy = gy - y0
    wm1 = wv - 1.0
    vx0 = ((x0 >= 0.0) & (x0 <= wm1)).astype(i32)
    vx1 = ((x0 >= -1.0) & (x0 <= wm1 - 1.0)).astype(i32)
    vy0 = ((y0 >= 0.0) & (y0 <= wm1)).astype(i32)
    vy1 = ((y0 >= -1.0) & (y0 <= wm1 - 1.0)).astype(i32)
    xc = jnp.clip(x0, 0.0, wm1)
    yc = jnp.clip(y0, 0.0, wm1)
    dx = jnp.clip((jnp.clip(x0 + 1.0, 0.0, wm1) - xc).astype(i32), 0, 1)
    dy = jnp.clip((jnp.clip(y0 + 1.0, 0.0, wm1) - yc).astype(i32), 0, 1)
    idx_i = jnp.clip((basev + yc * wv + xc).astype(i32), lo_ref[0], hi_ref[0])
    packed = (idx_i | (vx0 << 12) | (vx1 << 13) | (vy0 << 14) | (vy1 << 15)
              | (dx << 16) | (dy << 17))
    comps = (packed, lax.bitcast_convert_type(fx, i32),
             lax.bitcast_convert_type(fy, i32),
             lax.bitcast_convert_type(aw, i32))
    for cc, arr in enumerate(comps):
        for h in range(N_HEADS):
            pk_ref[0, h, :, pl.ds(cc * LP, LP)] = arr[:, h * LP:(h + 1) * LP]
    v = jnp.dot(val_ref[0], wv_ref[...], preferred_element_type=f32) + bv_ref[0]
    for h in range(N_HEADS):
        v_ref[0, h] = v[:, h * HEAD_DIM:(h + 1) * HEAD_DIM]


def _prep_call(bs, q_p, rb8, val_p, wcat, waw, wv, bcat, baw, bv, bxy, g,
               wvec, basev, lo_i, hi_i):
    grid = (bs, NQB)
    qmap = lambda b, i: (b, i, 0)
    full = lambda b, i: (0, 0)
    return pl.pallas_call(
        _prep_body,
        grid=grid,
        in_specs=[
            pl.BlockSpec((1, BQ, D_MODEL), qmap),
            pl.BlockSpec((1, BQ, 8), qmap),
            pl.BlockSpec((1, BQ, D_MODEL), qmap),
            pl.BlockSpec((D_MODEL, 256), full),
            pl.BlockSpec((D_MODEL, 128), full),
            pl.BlockSpec((D_MODEL, D_MODEL), full),
            pl.BlockSpec((1, 256), full),
            pl.BlockSpec((1, 128), full),
            pl.BlockSpec((1, D_MODEL), full),
            pl.BlockSpec((8, 256), full),
            pl.BlockSpec((128, 128), full),
            pl.BlockSpec((1, 128), full),
            pl.BlockSpec((1, 128), full),
            pl.BlockSpec((1, 128), full),
            pl.BlockSpec((1, 128), full),
        ],
        out_specs=[
            pl.BlockSpec((1, N_HEADS, BQ, HEAD_DIM), lambda b, i: (b, 0, i, 0)),
            pl.BlockSpec((1, N_HEADS, BQ, N_CMP * LP), lambda b, i: (b, 0, i, 0)),
        ],
        out_shape=[
            jax.ShapeDtypeStruct((bs, N_HEADS, NQ_PAD, HEAD_DIM), jnp.float32),
            jax.ShapeDtypeStruct((bs, N_HEADS, NQ_PAD, N_CMP * LP), jnp.int32),
        ],
    )(q_p, rb8, val_p, wcat, waw, wv, bcat, baw, bv, bxy, g, wvec, basev,
      lo_i, hi_i)


def _lane_bcast(vec, t):
    tv = jnp.full((16,), t, jnp.int32)
    dn = lax.GatherDimensionNumbers(offset_dims=(), collapsed_slice_dims=(0,),
                                    start_index_map=(0,))
    return lax.gather(vec, tv[:, None], dimension_numbers=dn, slice_sizes=(1,),
                      mode=lax.GatherScatterMode.PROMISE_IN_BOUNDS)


_W32 = tuple(int(w) * HEAD_DIM for w in LEVEL_WH)


def _sc_body(table_hbm, pk_hbm, out_hbm, table_v, in_v, out_v,
             sem_t, sem_i0, sem_i1, sem_o0, sem_o1):
    c = lax.axis_index("c")
    s = lax.axis_index("s")
    wid = s * 2 + c
    bh = wid % 16
    half = wid // 16
    iota_lo = lax.iota(jnp.int32, 16)
    qbase = half * (NQ_PAD // 2)
    sem_i = (sem_i0, sem_i1)
    sem_o = (sem_o0, sem_o1)
    lvl = lax.iota(jnp.int32, 16) >> 2
    w32 = jnp.where(lvl == 0, _W32[0],
                    jnp.where(lvl == 1, _W32[1],
                              jnp.where(lvl == 2, _W32[2], _W32[3])))

    def in_copy(k, b):
        q0 = qbase + k * CH
        return pltpu.make_async_copy(
            pk_hbm.at[bh, pl.ds(q0, CH), :], in_v.at[b], sem_i[b])

    def out_copy(k, b):
        q0 = qbase + k * CH
        return pltpu.make_async_copy(
            out_v.at[b], out_hbm.at[bh, pl.ds(q0, CH), :], sem_o[b])

    tdesc = pltpu.make_async_copy(table_hbm.at[bh], table_v, sem_t)
    tdesc.start()
    for b in (0, 1):
        in_copy(b, b).start()
    tdesc.wait()

    def outer(ci2, carry):
        for b in (0, 1):
            k = ci2 * 2 + b
            in_copy(k, b).wait()

            @pl.when(ci2 > 0)
            def _():
                out_copy(k - 2, b).wait()

            def qloop(qi, carry2):
                idxf = in_v[b, qi, pl.ds(0, LP)]
                fx = plsc.bitcast(in_v[b, qi, pl.ds(LP, LP)], jnp.float32)
                fy = plsc.bitcast(in_v[b, qi, pl.ds(2 * LP, LP)], jnp.float32)
                aw = plsc.bitcast(in_v[b, qi, pl.ds(3 * LP, LP)], jnp.float32)
                rows_a = (idxf & 0xFFF) * HEAD_DIM
                mx0 = ((idxf >> 12) & 1).astype(jnp.float32)
                mx1 = ((idxf >> 13) & 1).astype(jnp.float32)
                my0 = ((idxf >> 14) & 1).astype(jnp.float32)
                my1 = ((idxf >> 15) & 1).astype(jnp.float32)
                ax = (1.0 - fx) * mx0
                bxw = fx * mx1
                ay = (1.0 - fy) * my0 * aw
                byw = fy * my1 * aw
                dxw = ((idxf >> 16) & 1) << 5
                rows_c = rows_a + ((idxf >> 17) & 1) * w32
                groups = ((rows_a, ax * ay),
                          (rows_a + dxw, bxw * ay),
                          (rows_c, ax * byw),
                          (rows_c + dxw, bxw * byw))
                accs = []
                for gpair in (groups[:2], groups[2:]):
                    acc_lo = jnp.zeros((16,), jnp.float32)
                    acc_hi = jnp.zeros((16,), jnp.float32)
                    for rows, ws in gpair:
                        for t in range(16):
                            addr_lo = _lane_bcast(rows, t) + iota_lo
                            wb = _lane_bcast(ws, t)
                            lo = plsc.load_gather(table_v, [addr_lo])
                            hi = plsc.load_gather(table_v, [addr_lo + 16])
                            acc_lo = acc_lo + lo * wb
                            acc_hi = acc_hi + hi * wb
                    accs.append((acc_lo, acc_hi))
                (a0, h0), (a1, h1) = accs
                out_v[b, qi, pl.ds(0, 16)] = a0 + a1
                out_v[b, qi, pl.ds(16, 16)] = h0 + h1
                return carry2

            lax.fori_loop(0, CH, qloop, 0)
            out_copy(k, b).start()

            @pl.when(ci2 < N_OUTER - 1)
            def _():
                in_copy(k + 2, b).start()
        return carry

    lax.fori_loop(0, N_OUTER, outer, 0)
    for b in (0, 1):
        out_copy(N_CHUNK - 2 + b, b).wait()


def _sc_call(tables, packed):
    bs_h = tables.shape[0]
    mesh = plsc.VectorSubcoreMesh(core_axis_name="c", subcore_axis_name="s")
    f = pl.kernel(
        _sc_body,
        out_type=jax.ShapeDtypeStruct((bs_h, NQ_PAD, HEAD_DIM), jnp.float32),
        mesh=mesh,
        scratch_types=[
            pltpu.VMEM((NQ_PAD * HEAD_DIM,), jnp.float32),
            pltpu.VMEM((2, CH, N_CMP * LP), jnp.int32),
            pltpu.VMEM((2, CH, HEAD_DIM), jnp.float32),
            pltpu.SemaphoreType.DMA,
            pltpu.SemaphoreType.DMA,
            pltpu.SemaphoreType.DMA,
            pltpu.SemaphoreType.DMA,
            pltpu.SemaphoreType.DMA,
        ],
        compiler_params=pltpu.CompilerParams(needs_layout_passes=False),
    )
    return f(tables, packed)


def _proj_body(x_ref, w_ref, b_ref, o_ref):
    acc = jnp.broadcast_to(b_ref[0], (BQ, D_MODEL))
    for h in range(N_HEADS):
        acc = acc + jnp.dot(x_ref[0, h], w_ref[h],
                            preferred_element_type=jnp.float32)
    o_ref[0] = acc


def _proj_call(x, w, b):
    bs = x.shape[0]
    return pl.pallas_call(
        _proj_body,
        grid=(bs, NQB),
        in_specs=[
            pl.BlockSpec((1, N_HEADS, BQ, HEAD_DIM), lambda bb, i: (bb, 0, i, 0)),
            pl.BlockSpec((N_HEADS, HEAD_DIM, D_MODEL), lambda bb, i: (0, 0, 0)),
            pl.BlockSpec((1, D_MODEL), lambda bb, i: (0, 0)),
        ],
        out_specs=pl.BlockSpec((1, BQ, D_MODEL), lambda bb, i: (bb, i, 0)),
        out_shape=jax.ShapeDtypeStruct((bs, LEN_V, D_MODEL), jnp.float32),
    )(x, w, b)


def kernel(query, refer_bbox, value, value_shapes, W_so, b_so, W_aw, b_aw,
           W_v, b_v, W_o, b_o):
    bs, len_q, _ = query.shape
    rb8 = refer_bbox.reshape(bs, len_q, 8)

    wnp, basenp, bxnp, bynp, gnp = _lane_consts()
    wcat = jnp.concatenate([W_so[:, 0::2], W_so[:, 1::2]], axis=1)
    bcat = jnp.concatenate([b_so[0::2] - 0.5, b_so[1::2] - 0.5])[None]
    bxy = jnp.asarray(np.concatenate([bxnp * wnp, bynp * wnp], axis=1))
    wvec = jnp.asarray(wnp)[None]
    basev = jnp.asarray(basenp)[None]
    g = jnp.asarray(gnp)
    lo_i = jnp.asarray(basenp.astype(np.int32))[None]
    hi_i = jnp.asarray((basenp + wnp * wnp - 1).astype(np.int32))[None]

    v, pk = _prep_call(bs, query, rb8, value, wcat, W_aw, W_v,
                       bcat, b_aw[None], b_v[None], bxy, g, wvec, basev,
                       lo_i, hi_i)

    tables = v.reshape(bs * N_HEADS, NQ_PAD * HEAD_DIM)
    out_sc = _sc_call(tables, pk.reshape(bs * N_HEADS, NQ_PAD, N_CMP * LP))

    attn = out_sc.reshape(bs, N_HEADS, NQ_PAD, HEAD_DIM)
    return _proj_call(attn, W_o.reshape(N_HEADS, HEAD_DIM, D_MODEL), b_o[None])
